# baseline (speedup 1.0000x reference)
import functools

import jax
import jax.numpy as jnp
from jax import lax
from jax.experimental import pallas as pl
from jax.experimental.pallas import tpu as pltpu
from jax.experimental.pallas import tpu_sc as plsc


@functools.cache
def _make_gather(V, D, B):
    info = plsc.get_sparse_core_info()
    NC, NS = info.num_cores, info.num_subcores
    NW = NC * NS
    assert B % NW == 0
    b_per_w = B // NW
    CH = 1024
    assert b_per_w % CH == 0
    n_ch = b_per_w // CH
    mesh = plsc.VectorSubcoreMesh(core_axis_name="c", subcore_axis_name="s")

    @functools.partial(
        pl.kernel,
        mesh=mesh,
        out_type=jax.ShapeDtypeStruct((B, D), jnp.float32),
        scratch_types=[
            pltpu.VMEM((CH,), jnp.int32),
            pltpu.VMEM((CH, D), jnp.float32),
            pltpu.SemaphoreType.DMA,
        ],
        compiler_params=pltpu.CompilerParams(use_tc_tiling_on_sc=False),
    )
    def gather_kernel(idx_hbm, table_hbm, out_hbm, idx_v, rows_v, sem):
        wid = lax.axis_index("s") * NC + lax.axis_index("c")
        base = wid * b_per_w

        def body(i, carry):
            off = base + i * CH
            pltpu.sync_copy(idx_hbm.at[pl.ds(off, CH)], idx_v)
            pltpu.async_copy(table_hbm.at[idx_v], rows_v, sem).wait()
            pltpu.sync_copy(rows_v, out_hbm.at[pl.ds(off, CH)])
            return carry

        lax.fori_loop(0, n_ch, body, 0)

    return gather_kernel


def kernel(x, table):
    B, H = x.shape
    V, D = table.shape
    idx = x.reshape(B * H).astype(jnp.int32)
    out = _make_gather(V, D, B * H)(idx, table)
    return out.reshape(B, H, D)


# speedup vs baseline: 1.0187x; 1.0187x over previous
import functools

import jax
import jax.numpy as jnp
from jax import lax
from jax.experimental import pallas as pl
from jax.experimental.pallas import tpu as pltpu
from jax.experimental.pallas import tpu_sc as plsc

_NBUF = 2
_CH = 640


@functools.cache
def _make_gather(V, D, B):
    info = plsc.get_sparse_core_info()
    NC, NS = info.num_cores, info.num_subcores
    NW = NC * NS
    assert B % NW == 0
    b_per_w = B // NW
    assert b_per_w % (_CH * _NBUF) == 0
    n_ch = b_per_w // _CH
    mesh = plsc.VectorSubcoreMesh(core_axis_name="c", subcore_axis_name="s")

    @functools.partial(
        pl.kernel,
        mesh=mesh,
        out_type=jax.ShapeDtypeStruct((B, D), jnp.float32),
        scratch_types=[
            pltpu.VMEM((b_per_w,), jnp.int32),
            pltpu.VMEM((_NBUF, _CH, D), jnp.float32),
            pltpu.SemaphoreType.DMA,
            pltpu.SemaphoreType.DMA,
            pltpu.SemaphoreType.DMA,
            pltpu.SemaphoreType.DMA,
        ],
        compiler_params=pltpu.CompilerParams(use_tc_tiling_on_sc=False),
    )
    def gather_kernel(idx_hbm, table_hbm, out_hbm, idx_v, rows_v,
                      gsem0, gsem1, wsem0, wsem1):
        gsem = (gsem0, gsem1)
        wsem = (wsem0, wsem1)
        wid = lax.axis_index("s") * NC + lax.axis_index("c")
        base = wid * b_per_w
        pltpu.sync_copy(idx_hbm.at[pl.ds(base, b_per_w)], idx_v)

        def start_gather(i, b):
            off = pl.multiple_of(i * _CH, _CH)
            pltpu.async_copy(
                table_hbm.at[idx_v.at[pl.ds(off, _CH)]], rows_v.at[b], gsem[b])

        def wait_gather(b):
            pltpu.make_async_copy(
                table_hbm.at[idx_v.at[pl.ds(0, _CH)]], rows_v.at[b],
                gsem[b]).wait()

        def start_write(i, b):
            off = pl.multiple_of(base + i * _CH, _CH)
            pltpu.async_copy(rows_v.at[b], out_hbm.at[pl.ds(off, _CH)], wsem[b])

        def wait_write(b):
            pltpu.make_async_copy(
                rows_v.at[b], out_hbm.at[pl.ds(0, _CH)], wsem[b]).wait()

        start_gather(0, 0)
        start_gather(1, 1)
        wait_gather(0)
        start_write(0, 0)
        wait_gather(1)
        start_write(1, 1)

        def body(g, carry):
            for b in range(_NBUF):
                i = g * _NBUF + b
                wait_write(b)
                start_gather(i, b)
                wait_gather(b)
                start_write(i, b)
            return carry

        lax.fori_loop(1, n_ch // _NBUF, body, 0)
        wait_write(0)
        wait_write(1)

    return gather_kernel


def kernel(x, table):
    B, H = x.shape
    V, D = table.shape
    idx = x.reshape(B * H).astype(jnp.int32)
    out = _make_gather(V, D, B * H)(idx, table)
    return out.reshape(B, H, D)
